# baseline (speedup 1.0000x reference)
import functools

import jax
import jax.numpy as jnp
from jax import lax
from jax.experimental import pallas as pl
from jax.experimental.pallas import tpu as pltpu
from jax.experimental.pallas import tpu_sc as plsc

_NC, _NS = 2, 16
_CH = 128
_KPW = 80
_NB = 1024
_EB = 8000

_HIGH = jax.lax.Precision.HIGHEST


def _dot(a, b, precision=_HIGH):
    return jnp.dot(a, b, preferred_element_type=jnp.float32,
                   precision=precision)


def _pre1_body(x_ref, wl_ref, wr_ref, bl_ref, t_ref, xr_ref):
    x = x_ref[...]
    t_ref[...] = _dot(x, wl_ref[...])
    xr_ref[...] = _dot(x, wr_ref[...], jax.lax.Precision.DEFAULT) + bl_ref[...]


def _stats_body(p_ref, d_ref, xr_ref, pre_ref, s_ref, q_ref, *, n_real):
    i = pl.program_id(0)
    nb, h = xr_ref.shape
    deg = d_ref[:, 0:1] + d_ref[:, h:h + 1]
    inv = 1.0 / jnp.maximum(deg, 1.0)
    pre = (p_ref[:, :h] + p_ref[:, h:]) * inv + xr_ref[...]
    pre_ref[...] = pre
    rid = i * nb + lax.broadcasted_iota(jnp.int32, (nb, 1), 0)
    prem = jnp.where(rid < n_real, pre, 0.0)
    bs = jnp.sum(prem, axis=0, keepdims=True)
    bq = jnp.sum(prem * prem, axis=0, keepdims=True)

    @pl.when(i == 0)
    def _():
        s_ref[...] = bs
        q_ref[...] = bq

    @pl.when(i != 0)
    def _():
        s_ref[...] += bs
        q_ref[...] += bq


def _bn_relu(pre_ref, s_ref, q_ref, g_ref, be_ref, n):
    mu = s_ref[...] * (1.0 / n)
    var = q_ref[...] * (1.0 / n) - mu * mu
    h = (pre_ref[...] - mu) * lax.rsqrt(var + 1e-5) * g_ref[...] + be_ref[...]
    return jnp.maximum(h, 0.0)


def _apply1_body(pre_ref, s_ref, q_ref, g_ref, be_ref, wl_ref, wr_ref, bl_ref,
                 t_ref, xr_ref, *, n):
    h = _bn_relu(pre_ref, s_ref, q_ref, g_ref, be_ref, n)
    t_ref[...] = _dot(h, wl_ref[...])
    xr_ref[...] = _dot(h, wr_ref[...], jax.lax.Precision.DEFAULT) + bl_ref[...]


def _apply2_body(pre_ref, s_ref, q_ref, g_ref, be_ref, h_ref, *, n):
    h = _bn_relu(pre_ref, s_ref, q_ref, g_ref, be_ref, n)
    h_ref[...] = h.astype(jnp.bfloat16)


def _mlp_body(hsr_ref, ea_ref, w1sr_ref, w1e_ref, b1_ref,
              w2_ref, b2_ref, w3_ref, b3_ref, o_ref):
    p = jax.lax.Precision.DEFAULT
    bf = jnp.bfloat16
    z = _dot(hsr_ref[...], w1sr_ref[...], p)
    z += _dot(ea_ref[...], w1e_ref[...], p)
    z = jnp.maximum(z + b1_ref[...], 0.0)
    z = jnp.maximum(_dot(z.astype(bf), w2_ref[...], p) + b2_ref[...], 0.0)
    o_ref[...] = _dot(z.astype(bf), w3_ref[...], p) + b3_ref[...]


def _full(shape):
    return pl.BlockSpec(shape, lambda i: (0,) * len(shape))


def _rows(shape):
    return pl.BlockSpec(shape, lambda i: (i,) + (0,) * (len(shape) - 1))


def _sc_mesh():
    return plsc.VectorSubcoreMesh(core_axis_name="c", subcore_axis_name="s",
                                  num_cores=_NC, num_subcores=_NS)


_SC_PARAMS = pltpu.CompilerParams(use_tc_tiling_on_sc=False)


def _sc_segsum(table, srcr, dstr, z64, z16):
    n, d = table.shape
    nch, ch = srcr.shape
    with_deg = z16 is not None
    rpw = n // _NS
    assert nch == _NC * _NS * _KPW and rpw % 8 == 0 and n % _NS == 0
    nbuf, ahead = 4, 2
    ngrp = _KPW // nbuf

    out_type = [jax.ShapeDtypeStruct((n, 2 * d), jnp.float32)]
    scratch = [
        pltpu.VMEM((_KPW, ch), jnp.int32),
        pltpu.VMEM((nbuf, 1, ch), jnp.int32),
        pltpu.VMEM((nbuf, ch, d), jnp.float32),
        pltpu.VMEM_SHARED((n, d), jnp.float32),
        pltpu.SemaphoreType.DMA((2,)),
        pltpu.SemaphoreType.DMA((nbuf,)),
        pltpu.SemaphoreType.DMA((nbuf,)),
        pltpu.SemaphoreType.DMA((nbuf,)),
    ]
    if with_deg:
        out_type.append(jax.ShapeDtypeStruct((n, 2 * d), jnp.float32))
        scratch += [
            pltpu.VMEM((ch, 16), jnp.float32),
            pltpu.VMEM_SHARED((n, 16), jnp.float32),
            pltpu.SemaphoreType.DMA,
        ]

    def body(*refs):
        if with_deg:
            (tbl, sr, dr, z64r, z16r, out, dout, sidx, didx, rows, acc,
             isem, dsem, gsem, ssem, ones_v, dacc, osem) = refs
        else:
            (tbl, sr, dr, z64r, out, sidx, didx, rows, acc,
             isem, dsem, gsem, ssem) = refs
        cid = lax.axis_index("c")
        sid = lax.axis_index("s")
        c0 = pl.multiple_of((cid * _NS + sid) * _KPW, 8)
        idx_s = pltpu.async_copy(sr.at[pl.ds(c0, _KPW)], sidx, isem.at[0])

        row0 = pl.multiple_of(sid * rpw, 8)
        pltpu.sync_copy(z64r.at[pl.ds(row0, rpw)], acc.at[pl.ds(row0, rpw)])
        if with_deg:
            pltpu.sync_copy(z16r.at[pl.ds(row0, rpw)], dacc.at[pl.ds(row0, rpw)])

            @pl.loop(0, ch)
            def _(i):
                ones_v[i, :] = jnp.ones((16,), jnp.float32)

        idx_s.wait()
        plsc.subcore_barrier()

        def gath(k, b):
            pltpu.async_copy(tbl.at[sidx.at[k]], rows.at[b], gsem.at[b])
            pltpu.async_copy(dr.at[c0 + k], didx.at[b], dsem.at[b])

        def scat(k, b):
            pltpu.async_copy(rows.at[b], acc.at[didx.at[b, 0]], ssem.at[b],
                             add=True)
            if with_deg:
                pltpu.async_copy(ones_v, dacc.at[didx.at[b, 0]], osem,
                                 add=True)

        def wait_g(b):
            pltpu.make_async_copy(tbl.at[sidx.at[0]], rows.at[b],
                                  gsem.at[b]).wait()
            pltpu.make_async_copy(dr.at[0], didx.at[b], dsem.at[b]).wait()

        def wait_s(b):
            pltpu.make_async_copy(rows.at[b], acc.at[didx.at[b, 0]],
                                  ssem.at[b]).wait()

        for b in range(ahead):
            gath(b, b)

        @pl.loop(0, ngrp)
        def _(kk):
            for b in range(nbuf):
                k = kk * nbuf + b
                wait_g(b)
                scat(k, b)
                bg = (b + ahead) % nbuf
                if b < ahead:
                    @pl.when(kk >= 1)
                    def _():
                        wait_s(bg)
                    gath(k + ahead, bg)
                else:
                    @pl.when(kk <= ngrp - 2)
                    def _():
                        wait_s(bg)
                        gath(k + ahead, bg)

        for b in range(nbuf):
            wait_s(b)
        if with_deg:
            @pl.loop(0, _KPW)
            def _(k):
                pltpu.make_async_copy(ones_v, dacc.at[didx.at[0, 0]],
                                      osem).wait()

        plsc.subcore_barrier()
        col0 = cid * d
        pltpu.sync_copy(acc.at[pl.ds(row0, rpw)],
                        out.at[pl.ds(row0, rpw), pl.ds(col0, d)])
        if with_deg:
            pltpu.sync_copy(dacc.at[pl.ds(row0, rpw)],
                            dout.at[pl.ds(row0, rpw), pl.ds(col0, 16)])

    fn = pl.kernel(body, out_type=tuple(out_type), mesh=_sc_mesh(),
                   scratch_types=tuple(scratch), compiler_params=_SC_PARAMS)
    args = (table, srcr, dstr, z64) + ((z16,) if with_deg else ())
    return fn(*args)


def _sc_gather(table, srcr, dstr):
    n, d = table.shape
    nch, ch = srcr.shape
    assert nch == _NC * _NS * _KPW
    nbuf, ahead = 4, 2
    ngrp = _KPW // nbuf

    dt = table.dtype
    out_type = jax.ShapeDtypeStruct((nch * ch, 2 * d), dt)
    scratch = (
        pltpu.VMEM((_KPW, ch), jnp.int32),
        pltpu.VMEM((_KPW, ch), jnp.int32),
        pltpu.VMEM((nbuf, ch, d), dt),
        pltpu.VMEM((nbuf, ch, d), dt),
        pltpu.SemaphoreType.DMA((2,)),
        pltpu.SemaphoreType.DMA((nbuf,)),
        pltpu.SemaphoreType.DMA((nbuf,)),
        pltpu.SemaphoreType.DMA((nbuf,)),
        pltpu.SemaphoreType.DMA((nbuf,)),
    )

    def body(tbl, sr, dr, hsr, sidx, didx, rows_s, rows_d,
             isem, gs, gd, ws, wd):
        w = lax.axis_index("c") * _NS + lax.axis_index("s")
        c0 = pl.multiple_of(w * _KPW, 8)
        idx_s = pltpu.async_copy(sr.at[pl.ds(c0, _KPW)], sidx, isem.at[0])
        idx_d = pltpu.async_copy(dr.at[pl.ds(c0, _KPW)], didx, isem.at[1])
        idx_s.wait()
        idx_d.wait()

        def gath(k, b):
            pltpu.async_copy(tbl.at[sidx.at[k]], rows_s.at[b], gs.at[b])
            pltpu.async_copy(tbl.at[didx.at[k]], rows_d.at[b], gd.at[b])

        def write(k, b):
            e0 = pl.multiple_of((c0 + k) * ch, 8)
            pltpu.async_copy(rows_s.at[b],
                             hsr.at[pl.ds(e0, ch), pl.ds(0, d)], ws.at[b])
            pltpu.async_copy(rows_d.at[b],
                             hsr.at[pl.ds(e0, ch), pl.ds(d, d)], wd.at[b])

        def wait_g(b):
            pltpu.make_async_copy(tbl.at[sidx.at[0]], rows_s.at[b],
                                  gs.at[b]).wait()
            pltpu.make_async_copy(tbl.at[didx.at[0]], rows_d.at[b],
                                  gd.at[b]).wait()

        def wait_w(b):
            pltpu.make_async_copy(rows_s.at[b],
                                  hsr.at[pl.ds(0, ch), pl.ds(0, d)],
                                  ws.at[b]).wait()
            pltpu.make_async_copy(rows_d.at[b],
                                  hsr.at[pl.ds(0, ch), pl.ds(d, d)],
                                  wd.at[b]).wait()

        for b in range(ahead):
            gath(b, b)

        @pl.loop(0, ngrp)
        def _(kk):
            for b in range(nbuf):
                k = kk * nbuf + b
                wait_g(b)
                write(k, b)
                bg = (b + ahead) % nbuf
                if b < ahead:
                    @pl.when(kk >= 1)
                    def _():
                        wait_w(bg)
                    gath(k + ahead, bg)
                else:
                    @pl.when(kk <= ngrp - 2)
                    def _():
                        wait_w(bg)
                        gath(k + ahead, bg)

        for b in range(nbuf):
            wait_w(b)

    fn = pl.kernel(body, out_type=out_type, mesh=_sc_mesh(),
                   scratch_types=scratch, compiler_params=_SC_PARAMS)
    return fn(table, srcr, dstr)


def kernel(x, edge_index, edge_attr, Wl1, bl1, Wr1, g1, be1, Wl2, bl2, Wr2,
           g2, be2, W1, B1, W2, B2, W3, B3):
    n, df = x.shape
    e = edge_index.shape[1]
    h = Wl1.shape[0]
    de = edge_attr.shape[1]
    f32 = jnp.float32

    npad = -(-n // (_NS * 8)) * (_NS * 8)
    npad = -(-npad // _NB) * _NB
    epad = _NC * _NS * _KPW * _CH
    pe = epad - e
    padidx = n + jnp.arange(pe, dtype=jnp.int32) % (npad - n)
    srcp = jnp.concatenate([edge_index[0], padidx])
    dstp = jnp.concatenate([edge_index[1], padidx])
    srcr = srcp.reshape(epad // _CH, _CH)
    dstr = dstp.reshape(epad // _CH, _CH)
    dstr3 = dstp.reshape(epad // _CH, 1, _CH)
    xp = jnp.concatenate([x, jnp.zeros((npad - n, df), f32)])
    z64 = jnp.zeros((npad, h), f32)
    z16 = jnp.zeros((npad, 16), f32)
    row = lambda v: v.reshape(1, -1)
    gn = npad // _NB

    t1, xr1 = pl.pallas_call(
        _pre1_body,
        grid=(gn,),
        in_specs=[_rows((_NB, df)), _full((df, h)), _full((df, h)),
                  _full((1, h))],
        out_specs=[_rows((_NB, h)), _rows((_NB, h))],
        out_shape=[jax.ShapeDtypeStruct((npad, h), f32)] * 2,
    )(xp, Wl1.T, Wr1.T, row(bl1))

    p1, pdeg = _sc_segsum(t1, srcr, dstr3, z64, z16)

    stats_call = pl.pallas_call(
        functools.partial(_stats_body, n_real=n),
        grid=(gn,),
        in_specs=[_rows((_NB, 2 * h)), _rows((_NB, 2 * h)), _rows((_NB, h))],
        out_specs=[_rows((_NB, h)), _full((1, h)), _full((1, h))],
        out_shape=[jax.ShapeDtypeStruct((npad, h), f32),
                   jax.ShapeDtypeStruct((1, h), f32),
                   jax.ShapeDtypeStruct((1, h), f32)],
    )

    pre1, s1, q1 = stats_call(p1, pdeg, xr1)

    t2, xr2 = pl.pallas_call(
        functools.partial(_apply1_body, n=float(n)),
        grid=(gn,),
        in_specs=[_rows((_NB, h)), _full((1, h)), _full((1, h)),
                  _full((1, h)), _full((1, h)), _full((h, h)), _full((h, h)),
                  _full((1, h))],
        out_specs=[_rows((_NB, h)), _rows((_NB, h))],
        out_shape=[jax.ShapeDtypeStruct((npad, h), f32)] * 2,
    )(pre1, s1, q1, row(g1), row(be1), Wl2.T, Wr2.T, row(bl2))

    (p2,) = _sc_segsum(t2, srcr, dstr3, z64, None)

    pre2, s2, q2 = stats_call(p2, pdeg, xr2)

    h2 = pl.pallas_call(
        functools.partial(_apply2_body, n=float(n)),
        grid=(gn,),
        in_specs=[_rows((_NB, h)), _full((1, h)), _full((1, h)),
                  _full((1, h)), _full((1, h))],
        out_specs=_rows((_NB, h)),
        out_shape=jax.ShapeDtypeStruct((npad, h), jnp.bfloat16),
    )(pre2, s2, q2, row(g2), row(be2))

    hsr = _sc_gather(h2, srcr, dstr)

    bf = jnp.bfloat16
    w1sr = jnp.concatenate([W1[:, :h].T, W1[:, h + de:].T], axis=0)
    out = pl.pallas_call(
        _mlp_body,
        grid=(e // _EB,),
        in_specs=[_rows((_EB, 2 * h)), _rows((_EB, de)),
                  _full((2 * h, 128)), _full((de, 128)),
                  _full((1, 128)), _full((128, 64)), _full((1, 64)),
                  _full((64, 2)), _full((1, 2))],
        out_specs=_rows((_EB, 2)),
        out_shape=jax.ShapeDtypeStruct((e, 2), f32),
    )(hsr, edge_attr.astype(bf), w1sr.astype(bf), W1[:, h:h + de].T.astype(bf),
      row(B1), W2.T.astype(bf), row(B2), W3.T.astype(bf), row(B3))

    return out


# speedup vs baseline: 1.3020x; 1.3020x over previous
import functools

import jax
import jax.numpy as jnp
from jax import lax
from jax.experimental import pallas as pl
from jax.experimental.pallas import tpu as pltpu
from jax.experimental.pallas import tpu_sc as plsc

_NC, _NS = 2, 16
_CH = 128
_KPW = 80
_NB = 1024
_EB = 8000

_HIGH = jax.lax.Precision.HIGHEST


def _dot(a, b, precision=_HIGH):
    return jnp.dot(a, b, preferred_element_type=jnp.float32,
                   precision=precision)


def _pre1_body(x_ref, wl_ref, wr_ref, bl_ref, t_ref, xr_ref):
    x = x_ref[...]
    t_ref[...] = _dot(x, wl_ref[...])
    xr_ref[...] = _dot(x, wr_ref[...], jax.lax.Precision.DEFAULT) + bl_ref[...]


def _stats_body(p_ref, d_ref, xr_ref, pre_ref, s_ref, q_ref, *, n_real):
    i = pl.program_id(0)
    nb, h = xr_ref.shape
    deg = d_ref[:, 0:1] + d_ref[:, h:h + 1]
    inv = 1.0 / jnp.maximum(deg, 1.0)
    pre = (p_ref[:, :h] + p_ref[:, h:]) * inv + xr_ref[...]
    pre_ref[...] = pre
    rid = i * nb + lax.broadcasted_iota(jnp.int32, (nb, 1), 0)
    prem = jnp.where(rid < n_real, pre, 0.0)
    bs = jnp.sum(prem, axis=0, keepdims=True)
    bq = jnp.sum(prem * prem, axis=0, keepdims=True)

    @pl.when(i == 0)
    def _():
        s_ref[...] = bs
        q_ref[...] = bq

    @pl.when(i != 0)
    def _():
        s_ref[...] += bs
        q_ref[...] += bq


def _bn_relu(pre_ref, s_ref, q_ref, g_ref, be_ref, n):
    mu = s_ref[...] * (1.0 / n)
    var = q_ref[...] * (1.0 / n) - mu * mu
    h = (pre_ref[...] - mu) * lax.rsqrt(var + 1e-5) * g_ref[...] + be_ref[...]
    return jnp.maximum(h, 0.0)


def _apply1_body(pre_ref, s_ref, q_ref, g_ref, be_ref, wl_ref, wr_ref, bl_ref,
                 t_ref, xr_ref, *, n):
    h = _bn_relu(pre_ref, s_ref, q_ref, g_ref, be_ref, n)
    t_ref[...] = _dot(h, wl_ref[...])
    xr_ref[...] = _dot(h, wr_ref[...], jax.lax.Precision.DEFAULT) + bl_ref[...]


def _apply2_body(pre_ref, s_ref, q_ref, g_ref, be_ref, h_ref, *, n):
    h_ref[...] = _bn_relu(pre_ref, s_ref, q_ref, g_ref, be_ref, n)


def _mlp_body(hsr_ref, ea_ref, w1sr_ref, w1e_ref, b1_ref,
              w2_ref, b2_ref, w3_ref, b3_ref, o_ref):
    p = jax.lax.Precision.DEFAULT
    bf = jnp.bfloat16
    z = _dot(hsr_ref[...].astype(bf), w1sr_ref[...], p)
    z += _dot(ea_ref[...].astype(bf), w1e_ref[...], p)
    z = jnp.maximum(z + b1_ref[...], 0.0)
    z = jnp.maximum(_dot(z.astype(bf), w2_ref[...], p) + b2_ref[...], 0.0)
    o_ref[...] = _dot(z.astype(bf), w3_ref[...], p) + b3_ref[...]


def _full(shape):
    return pl.BlockSpec(shape, lambda i: (0,) * len(shape))


def _rows(shape):
    return pl.BlockSpec(shape, lambda i: (i,) + (0,) * (len(shape) - 1))


def _sc_mesh():
    return plsc.VectorSubcoreMesh(core_axis_name="c", subcore_axis_name="s",
                                  num_cores=_NC, num_subcores=_NS)


_SC_PARAMS = pltpu.CompilerParams(use_tc_tiling_on_sc=False)


def _sc_segsum(table, srcr, dstr, z64, z16):
    n, d = table.shape
    nch, ch = srcr.shape
    with_deg = z16 is not None
    rpw = n // _NS
    assert nch == _NC * _NS * _KPW and rpw % 8 == 0 and n % _NS == 0
    nbuf, ahead = 4, 2
    ngrp = _KPW // nbuf

    out_type = [jax.ShapeDtypeStruct((n, 2 * d), jnp.float32)]
    scratch = [
        pltpu.VMEM((_KPW, ch), jnp.int32),
        pltpu.VMEM((nbuf, 1, ch), jnp.int32),
        pltpu.VMEM((nbuf, ch, d), jnp.float32),
        pltpu.VMEM_SHARED((n, d), jnp.float32),
        pltpu.SemaphoreType.DMA((2,)),
        pltpu.SemaphoreType.DMA((nbuf,)),
        pltpu.SemaphoreType.DMA((nbuf,)),
        pltpu.SemaphoreType.DMA((nbuf,)),
    ]
    if with_deg:
        out_type.append(jax.ShapeDtypeStruct((n, 2 * d), jnp.float32))
        scratch += [
            pltpu.VMEM((ch, 16), jnp.float32),
            pltpu.VMEM_SHARED((n, 16), jnp.float32),
            pltpu.SemaphoreType.DMA,
        ]

    def body(*refs):
        if with_deg:
            (tbl, sr, dr, z64r, z16r, out, dout, sidx, didx, rows, acc,
             isem, dsem, gsem, ssem, ones_v, dacc, osem) = refs
        else:
            (tbl, sr, dr, z64r, out, sidx, didx, rows, acc,
             isem, dsem, gsem, ssem) = refs
        cid = lax.axis_index("c")
        sid = lax.axis_index("s")
        c0 = pl.multiple_of((cid * _NS + sid) * _KPW, 8)
        idx_s = pltpu.async_copy(sr.at[pl.ds(c0, _KPW)], sidx, isem.at[0])

        row0 = pl.multiple_of(sid * rpw, 8)
        pltpu.sync_copy(z64r.at[pl.ds(row0, rpw)], acc.at[pl.ds(row0, rpw)])
        if with_deg:
            pltpu.sync_copy(z16r.at[pl.ds(row0, rpw)], dacc.at[pl.ds(row0, rpw)])

            @pl.loop(0, ch)
            def _(i):
                ones_v[i, :] = jnp.ones((16,), jnp.float32)

        idx_s.wait()
        plsc.subcore_barrier()

        def gath(k, b):
            pltpu.async_copy(tbl.at[sidx.at[k]], rows.at[b], gsem.at[b])
            pltpu.async_copy(dr.at[c0 + k], didx.at[b], dsem.at[b])

        def scat(k, b):
            pltpu.async_copy(rows.at[b], acc.at[didx.at[b, 0]], ssem.at[b],
                             add=True)
            if with_deg:
                pltpu.async_copy(ones_v, dacc.at[didx.at[b, 0]], osem,
                                 add=True)

        def wait_g(b):
            pltpu.make_async_copy(tbl.at[sidx.at[0]], rows.at[b],
                                  gsem.at[b]).wait()
            pltpu.make_async_copy(dr.at[0], didx.at[b], dsem.at[b]).wait()

        def wait_s(b):
            pltpu.make_async_copy(rows.at[b], acc.at[didx.at[b, 0]],
                                  ssem.at[b]).wait()

        for b in range(ahead):
            gath(b, b)

        @pl.loop(0, ngrp)
        def _(kk):
            for b in range(nbuf):
                k = kk * nbuf + b
                wait_g(b)
                scat(k, b)
                bg = (b + ahead) % nbuf
                if b < ahead:
                    @pl.when(kk >= 1)
                    def _():
                        wait_s(bg)
                    gath(k + ahead, bg)
                else:
                    @pl.when(kk <= ngrp - 2)
                    def _():
                        wait_s(bg)
                        gath(k + ahead, bg)

        for b in range(nbuf):
            wait_s(b)
        if with_deg:
            @pl.loop(0, _KPW)
            def _(k):
                pltpu.make_async_copy(ones_v, dacc.at[didx.at[0, 0]],
                                      osem).wait()

        plsc.subcore_barrier()
        col0 = cid * d
        pltpu.sync_copy(acc.at[pl.ds(row0, rpw)],
                        out.at[pl.ds(row0, rpw), pl.ds(col0, d)])
        if with_deg:
            pltpu.sync_copy(dacc.at[pl.ds(row0, rpw)],
                            dout.at[pl.ds(row0, rpw), pl.ds(col0, 16)])

    fn = pl.kernel(body, out_type=tuple(out_type), mesh=_sc_mesh(),
                   scratch_types=tuple(scratch), compiler_params=_SC_PARAMS)
    args = (table, srcr, dstr, z64) + ((z16,) if with_deg else ())
    return fn(*args)


def _sc_gather(table, srcr, dstr):
    n, d = table.shape
    nch, ch = srcr.shape
    kpw = nch // (_NC * _NS)
    nbuf, ahead = 4, 2
    assert nch == _NC * _NS * kpw and kpw % nbuf == 0
    ngrp = kpw // nbuf

    dt = table.dtype
    out_type = jax.ShapeDtypeStruct((nch * ch, 2 * d), dt)
    scratch = (
        pltpu.VMEM((kpw, ch), jnp.int32),
        pltpu.VMEM((kpw, ch), jnp.int32),
        pltpu.VMEM((nbuf, ch, d), dt),
        pltpu.VMEM((nbuf, ch, d), dt),
        pltpu.SemaphoreType.DMA((2,)),
        pltpu.SemaphoreType.DMA((nbuf,)),
        pltpu.SemaphoreType.DMA((nbuf,)),
        pltpu.SemaphoreType.DMA((nbuf,)),
        pltpu.SemaphoreType.DMA((nbuf,)),
    )

    def body(tbl, sr, dr, hsr, sidx, didx, rows_s, rows_d,
             isem, gs, gd, ws, wd):
        w = lax.axis_index("c") * _NS + lax.axis_index("s")
        c0 = pl.multiple_of(w * kpw, 8)
        idx_s = pltpu.async_copy(sr.at[pl.ds(c0, kpw)], sidx, isem.at[0])
        idx_d = pltpu.async_copy(dr.at[pl.ds(c0, kpw)], didx, isem.at[1])
        idx_s.wait()
        idx_d.wait()

        def gath(k, b):
            pltpu.async_copy(tbl.at[sidx.at[k]], rows_s.at[b], gs.at[b])
            pltpu.async_copy(tbl.at[didx.at[k]], rows_d.at[b], gd.at[b])

        def write(k, b):
            e0 = pl.multiple_of((c0 + k) * ch, 8)
            pltpu.async_copy(rows_s.at[b],
                             hsr.at[pl.ds(e0, ch), pl.ds(0, d)], ws.at[b])
            pltpu.async_copy(rows_d.at[b],
                             hsr.at[pl.ds(e0, ch), pl.ds(d, d)], wd.at[b])

        def wait_g(b):
            pltpu.make_async_copy(tbl.at[sidx.at[0]], rows_s.at[b],
                                  gs.at[b]).wait()
            pltpu.make_async_copy(tbl.at[didx.at[0]], rows_d.at[b],
                                  gd.at[b]).wait()

        def wait_w(b):
            pltpu.make_async_copy(rows_s.at[b],
                                  hsr.at[pl.ds(0, ch), pl.ds(0, d)],
                                  ws.at[b]).wait()
            pltpu.make_async_copy(rows_d.at[b],
                                  hsr.at[pl.ds(0, ch), pl.ds(d, d)],
                                  wd.at[b]).wait()

        for b in range(ahead):
            gath(b, b)

        @pl.loop(0, ngrp)
        def _(kk):
            for b in range(nbuf):
                k = kk * nbuf + b
                wait_g(b)
                write(k, b)
                bg = (b + ahead) % nbuf
                if b < ahead:
                    @pl.when(kk >= 1)
                    def _():
                        wait_w(bg)
                    gath(k + ahead, bg)
                else:
                    @pl.when(kk <= ngrp - 2)
                    def _():
                        wait_w(bg)
                        gath(k + ahead, bg)

        for b in range(nbuf):
            wait_w(b)

    fn = pl.kernel(body, out_type=out_type, mesh=_sc_mesh(),
                   scratch_types=scratch, compiler_params=_SC_PARAMS)
    return fn(table, srcr, dstr)


def kernel(x, edge_index, edge_attr, Wl1, bl1, Wr1, g1, be1, Wl2, bl2, Wr2,
           g2, be2, W1, B1, W2, B2, W3, B3):
    n, df = x.shape
    e = edge_index.shape[1]
    h = Wl1.shape[0]
    de = edge_attr.shape[1]
    f32 = jnp.float32

    npad = -(-n // (_NS * 8)) * (_NS * 8)
    npad = -(-npad // _NB) * _NB
    epad = _NC * _NS * _KPW * _CH
    pe = epad - e
    eh = e // 2
    ehp = epad // 2
    padidx = n + jnp.arange(ehp - eh, dtype=jnp.int32) % (npad - n)
    halves = []
    for i in range(2):
        halves.append((
            jnp.concatenate([edge_index[0, i * eh:(i + 1) * eh], padidx]),
            jnp.concatenate([edge_index[1, i * eh:(i + 1) * eh], padidx]),
        ))
    srcp = jnp.concatenate([halves[0][0], halves[1][0]])
    dstp = jnp.concatenate([halves[0][1], halves[1][1]])
    srcr = srcp.reshape(epad // _CH, _CH)
    dstr3 = dstp.reshape(epad // _CH, 1, _CH)
    xp = jnp.concatenate([x, jnp.zeros((npad - n, df), f32)])
    z64 = jnp.zeros((npad, h), f32)
    z16 = jnp.zeros((npad, 16), f32)
    row = lambda v: v.reshape(1, -1)
    gn = npad // _NB

    t1, xr1 = pl.pallas_call(
        _pre1_body,
        grid=(gn,),
        in_specs=[_rows((_NB, df)), _full((df, h)), _full((df, h)),
                  _full((1, h))],
        out_specs=[_rows((_NB, h)), _rows((_NB, h))],
        out_shape=[jax.ShapeDtypeStruct((npad, h), f32)] * 2,
    )(xp, Wl1.T, Wr1.T, row(bl1))

    p1, pdeg = _sc_segsum(t1, srcr, dstr3, z64, z16)

    stats_call = pl.pallas_call(
        functools.partial(_stats_body, n_real=n),
        grid=(gn,),
        in_specs=[_rows((_NB, 2 * h)), _rows((_NB, 2 * h)), _rows((_NB, h))],
        out_specs=[_rows((_NB, h)), _full((1, h)), _full((1, h))],
        out_shape=[jax.ShapeDtypeStruct((npad, h), f32),
                   jax.ShapeDtypeStruct((1, h), f32),
                   jax.ShapeDtypeStruct((1, h), f32)],
    )

    pre1, s1, q1 = stats_call(p1, pdeg, xr1)

    t2, xr2 = pl.pallas_call(
        functools.partial(_apply1_body, n=float(n)),
        grid=(gn,),
        in_specs=[_rows((_NB, h)), _full((1, h)), _full((1, h)),
                  _full((1, h)), _full((1, h)), _full((h, h)), _full((h, h)),
                  _full((1, h))],
        out_specs=[_rows((_NB, h)), _rows((_NB, h))],
        out_shape=[jax.ShapeDtypeStruct((npad, h), f32)] * 2,
    )(pre1, s1, q1, row(g1), row(be1), Wl2.T, Wr2.T, row(bl2))

    (p2,) = _sc_segsum(t2, srcr, dstr3, z64, None)

    pre2, s2, q2 = stats_call(p2, pdeg, xr2)

    h2 = pl.pallas_call(
        functools.partial(_apply2_body, n=float(n)),
        grid=(gn,),
        in_specs=[_rows((_NB, h)), _full((1, h)), _full((1, h)),
                  _full((1, h)), _full((1, h))],
        out_specs=_rows((_NB, h)),
        out_shape=jax.ShapeDtypeStruct((npad, h), f32),
    )(pre2, s2, q2, row(g2), row(be2))

    bf = jnp.bfloat16
    w1sr = jnp.concatenate([W1[:, :h].T, W1[:, h + de:].T], axis=0)
    mlp_call = pl.pallas_call(
        _mlp_body,
        grid=(eh // _EB,),
        in_specs=[_rows((_EB, 2 * h)), _rows((_EB, de)),
                  _full((2 * h, 128)), _full((de, 128)),
                  _full((1, 128)), _full((128, 64)), _full((1, 64)),
                  _full((64, 2)), _full((1, 2))],
        out_specs=_rows((_EB, 2)),
        out_shape=jax.ShapeDtypeStruct((eh, 2), f32),
    )
    outs = []
    for i in range(2):
        sr_i = srcr[i * (ehp // _CH):(i + 1) * (ehp // _CH)]
        dr_i = dstp[i * ehp:(i + 1) * ehp].reshape(ehp // _CH, _CH)
        hsr = _sc_gather(h2, sr_i, dr_i)
        ea_i = edge_attr[i * eh:(i + 1) * eh]
        outs.append(mlp_call(
            hsr, ea_i, w1sr.astype(bf),
            W1[:, h:h + de].T.astype(bf), row(B1), W2.T.astype(bf),
            row(B2), W3.T.astype(bf), row(B3)))

    return jnp.concatenate(outs, axis=0)


# speedup vs baseline: 1.8393x; 1.4126x over previous
import functools

import jax
import jax.numpy as jnp
from jax import lax
from jax.experimental import pallas as pl
from jax.experimental.pallas import tpu as pltpu
from jax.experimental.pallas import tpu_sc as plsc

_NC, _NS = 2, 16
_CH = 128
_KPW = 80
_NB = 1024
_EB = 6400

_HIGH = jax.lax.Precision.HIGHEST


def _dot(a, b, precision=_HIGH):
    return jnp.dot(a, b, preferred_element_type=jnp.float32,
                   precision=precision)


def _pre1_body(x_ref, wl_ref, wr_ref, bl_ref, t_ref, xr_ref):
    x = x_ref[...]
    t_ref[...] = _dot(x, wl_ref[...])
    xr_ref[...] = _dot(x, wr_ref[...], jax.lax.Precision.DEFAULT) + bl_ref[...]


def _stats_body(p_ref, d_ref, xr_ref, pre_ref, s_ref, q_ref, *, n_real):
    i = pl.program_id(0)
    nb, h = xr_ref.shape
    deg = d_ref[:, 0:1] + d_ref[:, h:h + 1]
    inv = 1.0 / jnp.maximum(deg, 1.0)
    pre = (p_ref[:, :h] + p_ref[:, h:]) * inv + xr_ref[...]
    pre_ref[...] = pre
    rid = i * nb + lax.broadcasted_iota(jnp.int32, (nb, 1), 0)
    prem = jnp.where(rid < n_real, pre, 0.0)
    bs = jnp.sum(prem, axis=0, keepdims=True)
    bq = jnp.sum(prem * prem, axis=0, keepdims=True)

    @pl.when(i == 0)
    def _():
        s_ref[...] = bs
        q_ref[...] = bq

    @pl.when(i != 0)
    def _():
        s_ref[...] += bs
        q_ref[...] += bq


def _bn_relu(pre_ref, s_ref, q_ref, g_ref, be_ref, n):
    mu = s_ref[...] * (1.0 / n)
    var = q_ref[...] * (1.0 / n) - mu * mu
    h = (pre_ref[...] - mu) * lax.rsqrt(var + 1e-5) * g_ref[...] + be_ref[...]
    return jnp.maximum(h, 0.0)


def _apply1_body(pre_ref, s_ref, q_ref, g_ref, be_ref, wl_ref, wr_ref, bl_ref,
                 t_ref, xr_ref, *, n):
    h = _bn_relu(pre_ref, s_ref, q_ref, g_ref, be_ref, n)
    t_ref[...] = _dot(h, wl_ref[...])
    xr_ref[...] = _dot(h, wr_ref[...], jax.lax.Precision.DEFAULT) + bl_ref[...]


def _apply2_body(pre_ref, s_ref, q_ref, g_ref, be_ref, h_ref, *, n):
    h_ref[...] = _bn_relu(pre_ref, s_ref, q_ref, g_ref, be_ref, n)


def _mlp_body(hsr_ref, ea_ref, w1sr_ref, w1e_ref, b1_ref,
              w2_ref, b2_ref, w3_ref, b3_ref, o_ref):
    p = jax.lax.Precision.DEFAULT
    bf = jnp.bfloat16
    f32 = jnp.float32
    z = _dot(hsr_ref[...].astype(bf), w1sr_ref[...], p)
    z += lax.dot_general(ea_ref[...].astype(bf), w1e_ref[...],
                         (((0,), (0,)), ((), ())),
                         preferred_element_type=f32, precision=p)
    z = jnp.maximum(z + b1_ref[...], 0.0)
    z = jnp.maximum(_dot(z.astype(bf), w2_ref[...], p) + b2_ref[...], 0.0)
    o_ref[...] = lax.dot_general(w3_ref[...], z.astype(bf),
                                 (((0,), (1,)), ((), ())),
                                 preferred_element_type=f32,
                                 precision=p) + b3_ref[...]


def _full(shape):
    return pl.BlockSpec(shape, lambda i: (0,) * len(shape))


def _rows(shape):
    return pl.BlockSpec(shape, lambda i: (i,) + (0,) * (len(shape) - 1))


def _sc_mesh():
    return plsc.VectorSubcoreMesh(core_axis_name="c", subcore_axis_name="s",
                                  num_cores=_NC, num_subcores=_NS)


_SC_PARAMS = pltpu.CompilerParams(use_tc_tiling_on_sc=False)


def _sc_segsum(table, srcr, dstr, z64, z16):
    n, d = table.shape
    nch, ch = srcr.shape
    with_deg = z16 is not None
    rpw = n // _NS
    assert nch == _NC * _NS * _KPW and rpw % 8 == 0 and n % _NS == 0
    nbuf, ahead = 4, 2
    ngrp = _KPW // nbuf

    out_type = [jax.ShapeDtypeStruct((n, 2 * d), jnp.float32)]
    scratch = [
        pltpu.VMEM((_KPW, ch), jnp.int32),
        pltpu.VMEM((nbuf, 1, ch), jnp.int32),
        pltpu.VMEM((nbuf, ch, d), jnp.float32),
        pltpu.VMEM_SHARED((n, d), jnp.float32),
        pltpu.SemaphoreType.DMA((2,)),
        pltpu.SemaphoreType.DMA((nbuf,)),
        pltpu.SemaphoreType.DMA((nbuf,)),
        pltpu.SemaphoreType.DMA((nbuf,)),
    ]
    if with_deg:
        out_type.append(jax.ShapeDtypeStruct((n, 2 * d), jnp.float32))
        scratch += [
            pltpu.VMEM((ch, 16), jnp.float32),
            pltpu.VMEM_SHARED((n, 16), jnp.float32),
            pltpu.SemaphoreType.DMA,
        ]

    def body(*refs):
        if with_deg:
            (tbl, sr, dr, z64r, z16r, out, dout, sidx, didx, rows, acc,
             isem, dsem, gsem, ssem, ones_v, dacc, osem) = refs
        else:
            (tbl, sr, dr, z64r, out, sidx, didx, rows, acc,
             isem, dsem, gsem, ssem) = refs
        cid = lax.axis_index("c")
        sid = lax.axis_index("s")
        c0 = pl.multiple_of((cid * _NS + sid) * _KPW, 8)
        idx_s = pltpu.async_copy(sr.at[pl.ds(c0, _KPW)], sidx, isem.at[0])

        row0 = pl.multiple_of(sid * rpw, 8)
        pltpu.sync_copy(z64r.at[pl.ds(row0, rpw)], acc.at[pl.ds(row0, rpw)])
        if with_deg:
            pltpu.sync_copy(z16r.at[pl.ds(row0, rpw)], dacc.at[pl.ds(row0, rpw)])

            @pl.loop(0, ch)
            def _(i):
                ones_v[i, :] = jnp.ones((16,), jnp.float32)

        idx_s.wait()
        plsc.subcore_barrier()

        def gath(k, b):
            pltpu.async_copy(tbl.at[sidx.at[k]], rows.at[b], gsem.at[b])
            pltpu.async_copy(dr.at[c0 + k], didx.at[b], dsem.at[b])

        def scat(k, b):
            pltpu.async_copy(rows.at[b], acc.at[didx.at[b, 0]], ssem.at[b],
                             add=True)
            if with_deg:
                pltpu.async_copy(ones_v, dacc.at[didx.at[b, 0]], osem,
                                 add=True)

        def wait_g(b):
            pltpu.make_async_copy(tbl.at[sidx.at[0]], rows.at[b],
                                  gsem.at[b]).wait()
            pltpu.make_async_copy(dr.at[0], didx.at[b], dsem.at[b]).wait()

        def wait_s(b):
            pltpu.make_async_copy(rows.at[b], acc.at[didx.at[b, 0]],
                                  ssem.at[b]).wait()

        for b in range(ahead):
            gath(b, b)

        @pl.loop(0, ngrp)
        def _(kk):
            for b in range(nbuf):
                k = kk * nbuf + b
                wait_g(b)
                scat(k, b)
                bg = (b + ahead) % nbuf
                if b < ahead:
                    @pl.when(kk >= 1)
                    def _():
                        wait_s(bg)
                    gath(k + ahead, bg)
                else:
                    @pl.when(kk <= ngrp - 2)
                    def _():
                        wait_s(bg)
                        gath(k + ahead, bg)

        for b in range(nbuf):
            wait_s(b)
        if with_deg:
            @pl.loop(0, _KPW)
            def _(k):
                pltpu.make_async_copy(ones_v, dacc.at[didx.at[0, 0]],
                                      osem).wait()

        plsc.subcore_barrier()
        col0 = cid * d
        pltpu.sync_copy(acc.at[pl.ds(row0, rpw)],
                        out.at[pl.ds(row0, rpw), pl.ds(col0, d)])
        if with_deg:
            pltpu.sync_copy(dacc.at[pl.ds(row0, rpw)],
                            dout.at[pl.ds(row0, rpw), pl.ds(col0, 16)])

    fn = pl.kernel(body, out_type=tuple(out_type), mesh=_sc_mesh(),
                   scratch_types=tuple(scratch), compiler_params=_SC_PARAMS)
    args = (table, srcr, dstr, z64) + ((z16,) if with_deg else ())
    return fn(*args)


def _sc_gather(table, srcr, dstr):
    n, d = table.shape
    nch, ch = srcr.shape
    kpw = nch // (_NC * _NS)
    nbuf, ahead = 4, 2
    assert nch == _NC * _NS * kpw and kpw % nbuf == 0
    ngrp = kpw // nbuf

    dt = table.dtype
    out_type = jax.ShapeDtypeStruct((nch * ch, 2 * d), dt)
    scratch = (
        pltpu.VMEM((kpw, ch), jnp.int32),
        pltpu.VMEM((kpw, ch), jnp.int32),
        pltpu.VMEM((nbuf, ch, d), dt),
        pltpu.VMEM((nbuf, ch, d), dt),
        pltpu.SemaphoreType.DMA((2,)),
        pltpu.SemaphoreType.DMA((nbuf,)),
        pltpu.SemaphoreType.DMA((nbuf,)),
        pltpu.SemaphoreType.DMA((nbuf,)),
        pltpu.SemaphoreType.DMA((nbuf,)),
    )

    def body(tbl, sr, dr, hsr, sidx, didx, rows_s, rows_d,
             isem, gs, gd, ws, wd):
        w = lax.axis_index("c") * _NS + lax.axis_index("s")
        c0 = pl.multiple_of(w * kpw, 8)
        idx_s = pltpu.async_copy(sr.at[pl.ds(c0, kpw)], sidx, isem.at[0])
        idx_d = pltpu.async_copy(dr.at[pl.ds(c0, kpw)], didx, isem.at[1])
        idx_s.wait()
        idx_d.wait()

        def gath(k, b):
            pltpu.async_copy(tbl.at[sidx.at[k]], rows_s.at[b], gs.at[b])
            pltpu.async_copy(tbl.at[didx.at[k]], rows_d.at[b], gd.at[b])

        def write(k, b):
            e0 = pl.multiple_of((c0 + k) * ch, 8)
            pltpu.async_copy(rows_s.at[b],
                             hsr.at[pl.ds(e0, ch), pl.ds(0, d)], ws.at[b])
            pltpu.async_copy(rows_d.at[b],
                             hsr.at[pl.ds(e0, ch), pl.ds(d, d)], wd.at[b])

        def wait_g(b):
            pltpu.make_async_copy(tbl.at[sidx.at[0]], rows_s.at[b],
                                  gs.at[b]).wait()
            pltpu.make_async_copy(tbl.at[didx.at[0]], rows_d.at[b],
                                  gd.at[b]).wait()

        def wait_w(b):
            pltpu.make_async_copy(rows_s.at[b],
                                  hsr.at[pl.ds(0, ch), pl.ds(0, d)],
                                  ws.at[b]).wait()
            pltpu.make_async_copy(rows_d.at[b],
                                  hsr.at[pl.ds(0, ch), pl.ds(d, d)],
                                  wd.at[b]).wait()

        for b in range(ahead):
            gath(b, b)

        @pl.loop(0, ngrp)
        def _(kk):
            for b in range(nbuf):
                k = kk * nbuf + b
                wait_g(b)
                write(k, b)
                bg = (b + ahead) % nbuf
                if b < ahead:
                    @pl.when(kk >= 1)
                    def _():
                        wait_w(bg)
                    gath(k + ahead, bg)
                else:
                    @pl.when(kk <= ngrp - 2)
                    def _():
                        wait_w(bg)
                        gath(k + ahead, bg)

        for b in range(nbuf):
            wait_w(b)

    fn = pl.kernel(body, out_type=out_type, mesh=_sc_mesh(),
                   scratch_types=scratch, compiler_params=_SC_PARAMS)
    return fn(table, srcr, dstr)


def kernel(x, edge_index, edge_attr, Wl1, bl1, Wr1, g1, be1, Wl2, bl2, Wr2,
           g2, be2, W1, B1, W2, B2, W3, B3):
    n, df = x.shape
    e = edge_index.shape[1]
    h = Wl1.shape[0]
    de = edge_attr.shape[1]
    f32 = jnp.float32

    npad = -(-n // (_NS * 8)) * (_NS * 8)
    npad = -(-npad // _NB) * _NB
    epad = _NC * _NS * _KPW * _CH
    pe = epad - e
    eh = e // 2
    ehp = epad // 2
    padidx = n + jnp.arange(ehp - eh, dtype=jnp.int32) % (npad - n)
    halves = []
    for i in range(2):
        halves.append((
            jnp.concatenate([edge_index[0, i * eh:(i + 1) * eh], padidx]),
            jnp.concatenate([edge_index[1, i * eh:(i + 1) * eh], padidx]),
        ))
    srcp = jnp.concatenate([halves[0][0], halves[1][0]])
    dstp = jnp.concatenate([halves[0][1], halves[1][1]])
    srcr = srcp.reshape(epad // _CH, _CH)
    dstr3 = dstp.reshape(epad // _CH, 1, _CH)
    xp = jnp.concatenate([x, jnp.zeros((npad - n, df), f32)])
    z64 = jnp.zeros((npad, h), f32)
    z16 = jnp.zeros((npad, 16), f32)
    row = lambda v: v.reshape(1, -1)
    gn = npad // _NB

    t1, xr1 = pl.pallas_call(
        _pre1_body,
        grid=(gn,),
        in_specs=[_rows((_NB, df)), _full((df, h)), _full((df, h)),
                  _full((1, h))],
        out_specs=[_rows((_NB, h)), _rows((_NB, h))],
        out_shape=[jax.ShapeDtypeStruct((npad, h), f32)] * 2,
    )(xp, Wl1.T, Wr1.T, row(bl1))

    p1, pdeg = _sc_segsum(t1, srcr, dstr3, z64, z16)

    stats_call = pl.pallas_call(
        functools.partial(_stats_body, n_real=n),
        grid=(gn,),
        in_specs=[_rows((_NB, 2 * h)), _rows((_NB, 2 * h)), _rows((_NB, h))],
        out_specs=[_rows((_NB, h)), _full((1, h)), _full((1, h))],
        out_shape=[jax.ShapeDtypeStruct((npad, h), f32),
                   jax.ShapeDtypeStruct((1, h), f32),
                   jax.ShapeDtypeStruct((1, h), f32)],
    )

    pre1, s1, q1 = stats_call(p1, pdeg, xr1)

    t2, xr2 = pl.pallas_call(
        functools.partial(_apply1_body, n=float(n)),
        grid=(gn,),
        in_specs=[_rows((_NB, h)), _full((1, h)), _full((1, h)),
                  _full((1, h)), _full((1, h)), _full((h, h)), _full((h, h)),
                  _full((1, h))],
        out_specs=[_rows((_NB, h)), _rows((_NB, h))],
        out_shape=[jax.ShapeDtypeStruct((npad, h), f32)] * 2,
    )(pre1, s1, q1, row(g1), row(be1), Wl2.T, Wr2.T, row(bl2))

    (p2,) = _sc_segsum(t2, srcr, dstr3, z64, None)

    pre2, s2, q2 = stats_call(p2, pdeg, xr2)

    h2 = pl.pallas_call(
        functools.partial(_apply2_body, n=float(n)),
        grid=(gn,),
        in_specs=[_rows((_NB, h)), _full((1, h)), _full((1, h)),
                  _full((1, h)), _full((1, h))],
        out_specs=_rows((_NB, h)),
        out_shape=jax.ShapeDtypeStruct((npad, h), f32),
    )(pre2, s2, q2, row(g2), row(be2))

    bf = jnp.bfloat16
    w1sr = jnp.concatenate([W1[:, :h].T, W1[:, h + de:].T], axis=0)
    eaT = edge_attr.T
    col = pl.BlockSpec((de, _EB), lambda i: (0, i))
    mlp_call = pl.pallas_call(
        _mlp_body,
        grid=(eh // _EB,),
        in_specs=[_rows((_EB, 2 * h)), col,
                  _full((2 * h, 128)), _full((de, 128)),
                  _full((1, 128)), _full((128, 64)), _full((1, 64)),
                  _full((64, 2)), _full((2, 1))],
        out_specs=pl.BlockSpec((2, _EB), lambda i: (0, i)),
        out_shape=jax.ShapeDtypeStruct((2, eh), f32),
    )
    outs = []
    for i in range(2):
        sr_i = srcr[i * (ehp // _CH):(i + 1) * (ehp // _CH)]
        dr_i = dstp[i * ehp:(i + 1) * ehp].reshape(ehp // _CH, _CH)
        hsr = _sc_gather(h2, sr_i, dr_i)
        outs.append(mlp_call(
            hsr, eaT[:, i * eh:(i + 1) * eh], w1sr.astype(bf),
            W1[:, h:h + de].T.astype(bf), row(B1), W2.T.astype(bf),
            row(B2), W3.T.astype(bf), B3.reshape(-1, 1)))

    return jnp.concatenate(outs, axis=1).T


# speedup vs baseline: 1.8723x; 1.0180x over previous
import functools

import jax
import jax.numpy as jnp
from jax import lax
from jax.experimental import pallas as pl
from jax.experimental.pallas import tpu as pltpu
from jax.experimental.pallas import tpu_sc as plsc

_NC, _NS = 2, 16
_CH = 128
_KPW = 80
_NB = 1024
_EB = 16000

_HIGH = jax.lax.Precision.HIGHEST


def _dot(a, b, precision=_HIGH):
    return jnp.dot(a, b, preferred_element_type=jnp.float32,
                   precision=precision)


def _pre1_body(x_ref, wl_ref, wr_ref, bl_ref, t_ref, xr_ref):
    x = x_ref[...]
    t_ref[...] = _dot(x, wl_ref[...])
    xr_ref[...] = _dot(x, wr_ref[...], jax.lax.Precision.DEFAULT) + bl_ref[...]


def _stats_body(p_ref, d_ref, xr_ref, pre_ref, s_ref, q_ref, *, n_real):
    i = pl.program_id(0)
    nb, h = xr_ref.shape
    deg = d_ref[:, 0:1] + d_ref[:, h:h + 1]
    inv = 1.0 / jnp.maximum(deg, 1.0)
    pre = (p_ref[:, :h] + p_ref[:, h:]) * inv + xr_ref[...]
    pre_ref[...] = pre
    rid = i * nb + lax.broadcasted_iota(jnp.int32, (nb, 1), 0)
    prem = jnp.where(rid < n_real, pre, 0.0)
    bs = jnp.sum(prem, axis=0, keepdims=True)
    bq = jnp.sum(prem * prem, axis=0, keepdims=True)

    @pl.when(i == 0)
    def _():
        s_ref[...] = bs
        q_ref[...] = bq

    @pl.when(i != 0)
    def _():
        s_ref[...] += bs
        q_ref[...] += bq


def _bn_relu(pre_ref, s_ref, q_ref, g_ref, be_ref, n):
    mu = s_ref[...] * (1.0 / n)
    var = q_ref[...] * (1.0 / n) - mu * mu
    h = (pre_ref[...] - mu) * lax.rsqrt(var + 1e-5) * g_ref[...] + be_ref[...]
    return jnp.maximum(h, 0.0)


def _apply1_body(pre_ref, s_ref, q_ref, g_ref, be_ref, wl_ref, wr_ref, bl_ref,
                 t_ref, xr_ref, *, n):
    h = _bn_relu(pre_ref, s_ref, q_ref, g_ref, be_ref, n)
    t_ref[...] = _dot(h, wl_ref[...])
    xr_ref[...] = _dot(h, wr_ref[...], jax.lax.Precision.DEFAULT) + bl_ref[...]


def _apply2_body(pre_ref, s_ref, q_ref, g_ref, be_ref, h_ref, *, n):
    h_ref[...] = _bn_relu(pre_ref, s_ref, q_ref, g_ref, be_ref, n)


def _mlp_body(hsr_ref, ea_ref, w1sr_ref, w1e_ref, b1_ref,
              w2_ref, b2_ref, w3_ref, b3_ref, o_ref):
    p = jax.lax.Precision.DEFAULT
    bf = jnp.bfloat16
    f32 = jnp.float32
    z = _dot(hsr_ref[...].astype(bf), w1sr_ref[...], p)
    z += lax.dot_general(ea_ref[...].astype(bf), w1e_ref[...],
                         (((0,), (0,)), ((), ())),
                         preferred_element_type=f32, precision=p)
    z = jnp.maximum(z + b1_ref[...], 0.0)
    z = jnp.maximum(_dot(z.astype(bf), w2_ref[...], p) + b2_ref[...], 0.0)
    o_ref[...] = lax.dot_general(w3_ref[...], z.astype(bf),
                                 (((0,), (1,)), ((), ())),
                                 preferred_element_type=f32,
                                 precision=p) + b3_ref[...]


def _full(shape):
    return pl.BlockSpec(shape, lambda i: (0,) * len(shape))


def _rows(shape):
    return pl.BlockSpec(shape, lambda i: (i,) + (0,) * (len(shape) - 1))


def _sc_mesh():
    return plsc.VectorSubcoreMesh(core_axis_name="c", subcore_axis_name="s",
                                  num_cores=_NC, num_subcores=_NS)


_SC_PARAMS = pltpu.CompilerParams(use_tc_tiling_on_sc=False)


def _sc_segsum(table, srcr, dstr, z64, z16):
    n, d = table.shape
    nch, ch = srcr.shape
    with_deg = z16 is not None
    rpw = n // _NS
    assert nch == _NC * _NS * _KPW and rpw % 8 == 0 and n % _NS == 0
    nbuf, ahead = 4, 2
    ngrp = _KPW // nbuf

    out_type = [jax.ShapeDtypeStruct((n, 2 * d), jnp.float32)]
    scratch = [
        pltpu.VMEM((_KPW, ch), jnp.int32),
        pltpu.VMEM((nbuf, 1, ch), jnp.int32),
        pltpu.VMEM((nbuf, ch, d), jnp.float32),
        pltpu.VMEM_SHARED((n, d), jnp.float32),
        pltpu.SemaphoreType.DMA((2,)),
        pltpu.SemaphoreType.DMA((nbuf,)),
        pltpu.SemaphoreType.DMA((nbuf,)),
        pltpu.SemaphoreType.DMA((nbuf,)),
    ]
    if with_deg:
        out_type.append(jax.ShapeDtypeStruct((n, 2 * d), jnp.float32))
        scratch += [
            pltpu.VMEM((ch, 16), jnp.float32),
            pltpu.VMEM_SHARED((n, 16), jnp.float32),
            pltpu.SemaphoreType.DMA,
        ]

    def body(*refs):
        if with_deg:
            (tbl, sr, dr, z64r, z16r, out, dout, sidx, didx, rows, acc,
             isem, dsem, gsem, ssem, ones_v, dacc, osem) = refs
        else:
            (tbl, sr, dr, z64r, out, sidx, didx, rows, acc,
             isem, dsem, gsem, ssem) = refs
        cid = lax.axis_index("c")
        sid = lax.axis_index("s")
        c0 = pl.multiple_of((cid * _NS + sid) * _KPW, 8)
        idx_s = pltpu.async_copy(sr.at[pl.ds(c0, _KPW)], sidx, isem.at[0])

        row0 = pl.multiple_of(sid * rpw, 8)
        pltpu.sync_copy(z64r.at[pl.ds(row0, rpw)], acc.at[pl.ds(row0, rpw)])
        if with_deg:
            pltpu.sync_copy(z16r.at[pl.ds(row0, rpw)], dacc.at[pl.ds(row0, rpw)])

            @pl.loop(0, ch)
            def _(i):
                ones_v[i, :] = jnp.ones((16,), jnp.float32)

        idx_s.wait()
        plsc.subcore_barrier()

        def gath(k, b):
            pltpu.async_copy(tbl.at[sidx.at[k]], rows.at[b], gsem.at[b])
            pltpu.async_copy(dr.at[c0 + k], didx.at[b], dsem.at[b])

        def scat(k, b):
            pltpu.async_copy(rows.at[b], acc.at[didx.at[b, 0]], ssem.at[b],
                             add=True)
            if with_deg:
                pltpu.async_copy(ones_v, dacc.at[didx.at[b, 0]], osem,
                                 add=True)

        def wait_g(b):
            pltpu.make_async_copy(tbl.at[sidx.at[0]], rows.at[b],
                                  gsem.at[b]).wait()
            pltpu.make_async_copy(dr.at[0], didx.at[b], dsem.at[b]).wait()

        def wait_s(b):
            pltpu.make_async_copy(rows.at[b], acc.at[didx.at[b, 0]],
                                  ssem.at[b]).wait()

        for b in range(ahead):
            gath(b, b)

        @pl.loop(0, ngrp)
        def _(kk):
            for b in range(nbuf):
                k = kk * nbuf + b
                wait_g(b)
                scat(k, b)
                bg = (b + ahead) % nbuf
                if b < ahead:
                    @pl.when(kk >= 1)
                    def _():
                        wait_s(bg)
                    gath(k + ahead, bg)
                else:
                    @pl.when(kk <= ngrp - 2)
                    def _():
                        wait_s(bg)
                        gath(k + ahead, bg)

        for b in range(nbuf):
            wait_s(b)
        if with_deg:
            @pl.loop(0, _KPW)
            def _(k):
                pltpu.make_async_copy(ones_v, dacc.at[didx.at[0, 0]],
                                      osem).wait()

        plsc.subcore_barrier()
        col0 = cid * d
        pltpu.sync_copy(acc.at[pl.ds(row0, rpw)],
                        out.at[pl.ds(row0, rpw), pl.ds(col0, d)])
        if with_deg:
            pltpu.sync_copy(dacc.at[pl.ds(row0, rpw)],
                            dout.at[pl.ds(row0, rpw), pl.ds(col0, 16)])

    fn = pl.kernel(body, out_type=tuple(out_type), mesh=_sc_mesh(),
                   scratch_types=tuple(scratch), compiler_params=_SC_PARAMS)
    args = (table, srcr, dstr, z64) + ((z16,) if with_deg else ())
    return fn(*args)


def _sc_gather(table, srcr, dstr):
    n, d = table.shape
    nch, ch = srcr.shape
    kpw = nch // (_NC * _NS)
    nbuf, ahead = 4, 2
    assert nch == _NC * _NS * kpw and kpw % nbuf == 0
    ngrp = kpw // nbuf

    dt = table.dtype
    out_type = jax.ShapeDtypeStruct((nch * ch, 2 * d), dt)
    scratch = (
        pltpu.VMEM((kpw, ch), jnp.int32),
        pltpu.VMEM((kpw, ch), jnp.int32),
        pltpu.VMEM((nbuf, ch, d), dt),
        pltpu.VMEM((nbuf, ch, d), dt),
        pltpu.SemaphoreType.DMA((2,)),
        pltpu.SemaphoreType.DMA((nbuf,)),
        pltpu.SemaphoreType.DMA((nbuf,)),
        pltpu.SemaphoreType.DMA((nbuf,)),
        pltpu.SemaphoreType.DMA((nbuf,)),
    )

    def body(tbl, sr, dr, hsr, sidx, didx, rows_s, rows_d,
             isem, gs, gd, ws, wd):
        w = lax.axis_index("c") * _NS + lax.axis_index("s")
        c0 = pl.multiple_of(w * kpw, 8)
        idx_s = pltpu.async_copy(sr.at[pl.ds(c0, kpw)], sidx, isem.at[0])
        idx_d = pltpu.async_copy(dr.at[pl.ds(c0, kpw)], didx, isem.at[1])
        idx_s.wait()
        idx_d.wait()

        def gath(k, b):
            pltpu.async_copy(tbl.at[sidx.at[k]], rows_s.at[b], gs.at[b])
            pltpu.async_copy(tbl.at[didx.at[k]], rows_d.at[b], gd.at[b])

        def write(k, b):
            e0 = pl.multiple_of((c0 + k) * ch, 8)
            pltpu.async_copy(rows_s.at[b],
                             hsr.at[pl.ds(e0, ch), pl.ds(0, d)], ws.at[b])
            pltpu.async_copy(rows_d.at[b],
                             hsr.at[pl.ds(e0, ch), pl.ds(d, d)], wd.at[b])

        def wait_g(b):
            pltpu.make_async_copy(tbl.at[sidx.at[0]], rows_s.at[b],
                                  gs.at[b]).wait()
            pltpu.make_async_copy(tbl.at[didx.at[0]], rows_d.at[b],
                                  gd.at[b]).wait()

        def wait_w(b):
            pltpu.make_async_copy(rows_s.at[b],
                                  hsr.at[pl.ds(0, ch), pl.ds(0, d)],
                                  ws.at[b]).wait()
            pltpu.make_async_copy(rows_d.at[b],
                                  hsr.at[pl.ds(0, ch), pl.ds(d, d)],
                                  wd.at[b]).wait()

        for b in range(ahead):
            gath(b, b)

        @pl.loop(0, ngrp)
        def _(kk):
            for b in range(nbuf):
                k = kk * nbuf + b
                wait_g(b)
                write(k, b)
                bg = (b + ahead) % nbuf
                if b < ahead:
                    @pl.when(kk >= 1)
                    def _():
                        wait_w(bg)
                    gath(k + ahead, bg)
                else:
                    @pl.when(kk <= ngrp - 2)
                    def _():
                        wait_w(bg)
                        gath(k + ahead, bg)

        for b in range(nbuf):
            wait_w(b)

    fn = pl.kernel(body, out_type=out_type, mesh=_sc_mesh(),
                   scratch_types=scratch, compiler_params=_SC_PARAMS)
    return fn(table, srcr, dstr)


def kernel(x, edge_index, edge_attr, Wl1, bl1, Wr1, g1, be1, Wl2, bl2, Wr2,
           g2, be2, W1, B1, W2, B2, W3, B3):
    n, df = x.shape
    e = edge_index.shape[1]
    h = Wl1.shape[0]
    de = edge_attr.shape[1]
    f32 = jnp.float32

    npad = -(-n // (_NS * 8)) * (_NS * 8)
    npad = -(-npad // _NB) * _NB
    epad = _NC * _NS * _KPW * _CH
    pe = epad - e
    nsplit = 4
    eh = e // nsplit
    ehp = epad // nsplit
    padidx = n + jnp.arange(ehp - eh, dtype=jnp.int32) % (npad - n)
    halves = []
    for i in range(nsplit):
        halves.append((
            jnp.concatenate([edge_index[0, i * eh:(i + 1) * eh], padidx]),
            jnp.concatenate([edge_index[1, i * eh:(i + 1) * eh], padidx]),
        ))
    srcp = jnp.concatenate([p[0] for p in halves])
    dstp = jnp.concatenate([p[1] for p in halves])
    srcr = srcp.reshape(epad // _CH, _CH)
    dstr3 = dstp.reshape(epad // _CH, 1, _CH)
    xp = jnp.concatenate([x, jnp.zeros((npad - n, df), f32)])
    z64 = jnp.zeros((npad, h), f32)
    z16 = jnp.zeros((npad, 16), f32)
    row = lambda v: v.reshape(1, -1)
    gn = npad // _NB

    t1, xr1 = pl.pallas_call(
        _pre1_body,
        grid=(gn,),
        in_specs=[_rows((_NB, df)), _full((df, h)), _full((df, h)),
                  _full((1, h))],
        out_specs=[_rows((_NB, h)), _rows((_NB, h))],
        out_shape=[jax.ShapeDtypeStruct((npad, h), f32)] * 2,
    )(xp, Wl1.T, Wr1.T, row(bl1))

    p1, pdeg = _sc_segsum(t1, srcr, dstr3, z64, z16)

    stats_call = pl.pallas_call(
        functools.partial(_stats_body, n_real=n),
        grid=(gn,),
        in_specs=[_rows((_NB, 2 * h)), _rows((_NB, 2 * h)), _rows((_NB, h))],
        out_specs=[_rows((_NB, h)), _full((1, h)), _full((1, h))],
        out_shape=[jax.ShapeDtypeStruct((npad, h), f32),
                   jax.ShapeDtypeStruct((1, h), f32),
                   jax.ShapeDtypeStruct((1, h), f32)],
    )

    pre1, s1, q1 = stats_call(p1, pdeg, xr1)

    t2, xr2 = pl.pallas_call(
        functools.partial(_apply1_body, n=float(n)),
        grid=(gn,),
        in_specs=[_rows((_NB, h)), _full((1, h)), _full((1, h)),
                  _full((1, h)), _full((1, h)), _full((h, h)), _full((h, h)),
                  _full((1, h))],
        out_specs=[_rows((_NB, h)), _rows((_NB, h))],
        out_shape=[jax.ShapeDtypeStruct((npad, h), f32)] * 2,
    )(pre1, s1, q1, row(g1), row(be1), Wl2.T, Wr2.T, row(bl2))

    (p2,) = _sc_segsum(t2, srcr, dstr3, z64, None)

    pre2, s2, q2 = stats_call(p2, pdeg, xr2)

    h2 = pl.pallas_call(
        functools.partial(_apply2_body, n=float(n)),
        grid=(gn,),
        in_specs=[_rows((_NB, h)), _full((1, h)), _full((1, h)),
                  _full((1, h)), _full((1, h))],
        out_specs=_rows((_NB, h)),
        out_shape=jax.ShapeDtypeStruct((npad, h), f32),
    )(pre2, s2, q2, row(g2), row(be2))

    bf = jnp.bfloat16
    w1sr = jnp.concatenate([W1[:, :h].T, W1[:, h + de:].T], axis=0)
    eaT = edge_attr.T
    col = pl.BlockSpec((de, _EB), lambda i: (0, i))
    mlp_call = pl.pallas_call(
        _mlp_body,
        grid=(eh // _EB,),
        in_specs=[_rows((_EB, 2 * h)), col,
                  _full((2 * h, 128)), _full((de, 128)),
                  _full((1, 128)), _full((128, 64)), _full((1, 64)),
                  _full((64, 2)), _full((2, 1))],
        out_specs=pl.BlockSpec((2, _EB), lambda i: (0, i)),
        out_shape=jax.ShapeDtypeStruct((2, eh), f32),
    )
    outs = []
    for i in range(nsplit):
        sr_i = srcr[i * (ehp // _CH):(i + 1) * (ehp // _CH)]
        dr_i = dstp[i * ehp:(i + 1) * ehp].reshape(ehp // _CH, _CH)
        hsr = _sc_gather(h2, sr_i, dr_i)
        outs.append(mlp_call(
            hsr, eaT[:, i * eh:(i + 1) * eh], w1sr.astype(bf),
            W1[:, h:h + de].T.astype(bf), row(B1), W2.T.astype(bf),
            row(B2), W3.T.astype(bf), B3.reshape(-1, 1)))

    return jnp.concatenate(outs, axis=1).T
